# baseline (speedup 1.0000x reference)
import jax
import jax.numpy as jnp
from jax.experimental import pallas as pl
from jax.experimental.pallas import tpu as pltpu

_BF = jnp.bfloat16
_F32 = jnp.float32


def _bmm(a, b):
    return jax.lax.dot_general(a.astype(_BF), b.astype(_BF),
                               (((1,), (0,)), ((), ())),
                               preferred_element_type=_F32)


def _bmm16(a, b):
    return jax.lax.dot_general(a, b, (((1,), (0,)), ((), ())),
                               preferred_element_type=_F32)


def _btmm16(a, b):
    return jax.lax.dot_general(a, b, (((0,), (0,)), ((), ())),
                               preferred_element_type=_F32)


def _gcn_kernel(adj_ref, feat_ref, wgc1_ref, wgc2_ref, g_ref,
                adjbf_ref, t1_ref, h1_ref, t2_ref, h2_ref):
    p = pl.program_id(0)
    j = pl.program_id(1)
    bm = adj_ref.shape[0]

    @pl.when(p == 0)
    def _():
        @pl.when(j == 0)
        def _():
            t1_ref[...] = _bmm(feat_ref[...], wgc1_ref[...]).astype(_BF)
        adjb = adj_ref[...].astype(_BF)
        adjbf_ref[pl.ds(j * bm, bm), :] = adjb
        h1_ref[pl.ds(j * bm, bm), :] = jnp.tanh(
            _bmm16(adjb, t1_ref[...])).astype(_BF)

    @pl.when(p == 1)
    def _():
        @pl.when(j == 0)
        def _():
            t2_ref[...] = _bmm(h1_ref[...], wgc2_ref[...]).astype(_BF)
        h2_ref[pl.ds(j * bm, bm), :] = jnp.maximum(
            _bmm16(adjbf_ref[pl.ds(j * bm, bm), :], t2_ref[...]),
            0.0).astype(_BF)

    @pl.when(p == 2)
    def _():
        g_ref[...] = _bmm16(adjbf_ref[pl.ds(j * bm, bm), :], h2_ref[...])


def _mid_kernel(g_ref, wl_ref, wm_ref, noise_ref, feat_ref, wmul_ref,
                bmul_ref, wd1_ref, bd1_ref, wd2_ref, bd2_ref,
                wdm_ref, bdm_ref, sf_ref, out_ref,
                acc_ref, featbf_ref, ghi_ref, glo_ref):
    j = pl.program_id(0)

    @pl.when(j == 0)
    def _():
        featbf_ref[...] = feat_ref[...].astype(_BF)
        g = g_ref[...]
        ghi = g.astype(_BF)
        ghi_ref[...] = ghi
        glo_ref[...] = (g - ghi.astype(_F32)).astype(_BF)

    ghi = ghi_ref[...]
    glo = glo_ref[...]
    wl = wl_ref[...].astype(_BF)
    wm = wm_ref[...].astype(_BF)
    zlog = _bmm16(ghi, wl) + _bmm16(glo, wl)
    zmean = _bmm16(ghi, wm) + _bmm16(glo, wm)
    z = zmean + noise_ref[...] * jnp.exp(zlog)
    p = _btmm16(featbf_ref[...], z.astype(_BF))
    contrib = _bmm(p, wmul_ref[...])

    @pl.when(j == 0)
    def _():
        acc_ref[...] = contrib

    @pl.when(j > 0)
    def _():
        acc_ref[...] += contrib

    @pl.when(j == pl.num_programs(0) - 1)
    def _():
        ei = jnp.maximum(acc_ref[...] + bmul_ref[...], 0.0)
        e1 = jnp.maximum(_bmm(ei, wd1_ref[...]) + bd1_ref[...], 0.0)
        e2 = jnp.maximum(_bmm(e1, wd2_ref[...]) + bd2_ref[...], 0.0)
        zm = jnp.clip(jnp.exp(_bmm(e2, wdm_ref[...]) + bdm_ref[...]),
                      1e-5, 1e6)
        out_ref[...] = zm * sf_ref[...]


def kernel(adj, features, size_factors, noise,
           W_gc1, W_gc2, W_mean, W_logstd, W_mul, b_mul,
           W_d1, b_d1, W_d2, b_d2, W_pi, b_pi, W_disp, b_disp, W_dm, b_dm):
    n, f = features.shape
    h1d = W_gc1.shape[1]
    h2d = W_gc2.shape[1]

    bm = 1024
    nrb = n // bm

    g = pl.pallas_call(
        _gcn_kernel,
        grid=(3, nrb),
        in_specs=[
            pl.BlockSpec((bm, n), lambda p, j: (jnp.where(p == 0, j, nrb - 1), 0)),
            pl.BlockSpec((n, f), lambda p, j: (0, 0)),
            pl.BlockSpec((f, h1d), lambda p, j: (0, 0)),
            pl.BlockSpec((h1d, h2d), lambda p, j: (0, 0)),
        ],
        out_specs=pl.BlockSpec((bm, h2d), lambda p, j: (j, 0)),
        out_shape=jax.ShapeDtypeStruct((n, h2d), _F32),
        scratch_shapes=[
            pltpu.VMEM((n, n), _BF),
            pltpu.VMEM((n, h1d), _BF),
            pltpu.VMEM((n, h1d), _BF),
            pltpu.VMEM((n, h2d), _BF),
            pltpu.VMEM((n, h2d), _BF),
        ],
        compiler_params=pltpu.CompilerParams(
            dimension_semantics=("arbitrary", "arbitrary")),
    )(adj, features, W_gc1, W_gc2)

    bn = 256
    nblk = n // bn
    out = pl.pallas_call(
        _mid_kernel,
        grid=(nblk,),
        in_specs=[
            pl.BlockSpec((n, h2d), lambda j: (0, 0)),
            pl.BlockSpec((h2d, bn), lambda j: (0, j)),
            pl.BlockSpec((h2d, bn), lambda j: (0, j)),
            pl.BlockSpec((n, bn), lambda j: (0, j)),
            pl.BlockSpec((n, f), lambda j: (0, 0)),
            pl.BlockSpec((bn, n), lambda j: (j, 0)),
            pl.BlockSpec((1, n), lambda j: (0, 0)),
            pl.BlockSpec((n, h1d), lambda j: (0, 0)),
            pl.BlockSpec((1, h1d), lambda j: (0, 0)),
            pl.BlockSpec((h1d, h2d), lambda j: (0, 0)),
            pl.BlockSpec((1, h2d), lambda j: (0, 0)),
            pl.BlockSpec((h2d, n), lambda j: (0, 0)),
            pl.BlockSpec((1, n), lambda j: (0, 0)),
            pl.BlockSpec((f, 1), lambda j: (0, 0)),
        ],
        out_specs=pl.BlockSpec((f, n), lambda j: (0, 0)),
        out_shape=jax.ShapeDtypeStruct((f, n), _F32),
        scratch_shapes=[
            pltpu.VMEM((f, n), _F32),
            pltpu.VMEM((n, f), _BF),
            pltpu.VMEM((n, h2d), _BF),
            pltpu.VMEM((n, h2d), _BF),
        ],
        compiler_params=pltpu.CompilerParams(
            dimension_semantics=("arbitrary",)),
    )(g, W_logstd, W_mean, noise, features, W_mul,
      b_mul.reshape(1, -1), W_d1, b_d1.reshape(1, -1),
      W_d2, b_d2.reshape(1, -1), W_dm, b_dm.reshape(1, -1),
      size_factors.reshape(-1, 1))
    return out


# speedup vs baseline: 1.1783x; 1.1783x over previous
import jax
import jax.numpy as jnp
from jax.experimental import pallas as pl
from jax.experimental.pallas import tpu as pltpu

_BF = jnp.bfloat16
_F32 = jnp.float32


def _bmm(a, b):
    return jax.lax.dot_general(a.astype(_BF), b.astype(_BF),
                               (((1,), (0,)), ((), ())),
                               preferred_element_type=_F32)


def _bmm16(a, b):
    return jax.lax.dot_general(a, b, (((1,), (0,)), ((), ())),
                               preferred_element_type=_F32)


def _btmm16(a, b):
    return jax.lax.dot_general(a, b, (((0,), (0,)), ((), ())),
                               preferred_element_type=_F32)


def _gcn_kernel(adj_ref, feat_ref, wgc1_ref, wgc2_ref, g_ref,
                adjbf_ref, t1_ref, h1_ref, t2_ref, h2_ref):
    p = pl.program_id(0)
    j = pl.program_id(1)
    bm = adj_ref.shape[0]

    @pl.when(p == 0)
    def _():
        @pl.when(j == 0)
        def _():
            t1_ref[...] = _bmm(feat_ref[...], wgc1_ref[...]).astype(_BF)
        adjb = adj_ref[...].astype(_BF)
        adjbf_ref[pl.ds(j * bm, bm), :] = adjb
        h1_ref[pl.ds(j * bm, bm), :] = jnp.tanh(
            _bmm16(adjb, t1_ref[...])).astype(_BF)

    @pl.when(p == 1)
    def _():
        @pl.when(j == 0)
        def _():
            t2_ref[...] = _bmm(h1_ref[...], wgc2_ref[...]).astype(_BF)
        h2_ref[pl.ds(j * bm, bm), :] = jnp.maximum(
            _bmm16(adjbf_ref[pl.ds(j * bm, bm), :], t2_ref[...]),
            0.0).astype(_BF)

    @pl.when(p == 2)
    def _():
        g_ref[...] = _bmm16(adjbf_ref[pl.ds(j * bm, bm), :], h2_ref[...])


def _mid_kernel(g_ref, wl_ref, wm_ref, noise_ref, feat_ref, wmul_ref,
                bmul_ref, wd1_ref, bd1_ref, wd2_ref, bd2_ref,
                wdm_ref, bdm_ref, sf_ref, out_ref,
                acc_ref, featbf_ref, ghi_ref, glo_ref):
    j = pl.program_id(0)

    @pl.when(j == 0)
    def _():
        featbf_ref[...] = feat_ref[...].astype(_BF)
        g = g_ref[...]
        ghi = g.astype(_BF)
        ghi_ref[...] = ghi
        glo_ref[...] = (g - ghi.astype(_F32)).astype(_BF)

    ghi = ghi_ref[...]
    glo = glo_ref[...]
    wl = wl_ref[...].astype(_BF)
    wm = wm_ref[...].astype(_BF)
    zlog = _bmm16(ghi, wl) + _bmm16(glo, wl)
    zmean = _bmm16(ghi, wm) + _bmm16(glo, wm)
    z = zmean + noise_ref[...] * jnp.exp(zlog)
    p = _btmm16(featbf_ref[...], z.astype(_BF))
    contrib = _bmm(p, wmul_ref[...])

    @pl.when(j == 0)
    def _():
        acc_ref[...] = contrib

    @pl.when(j > 0)
    def _():
        acc_ref[...] += contrib

    @pl.when(j == pl.num_programs(0) - 1)
    def _():
        ei = jnp.maximum(acc_ref[...] + bmul_ref[...], 0.0)
        e1 = jnp.maximum(_bmm(ei, wd1_ref[...]) + bd1_ref[...], 0.0)
        e2 = jnp.maximum(_bmm(e1, wd2_ref[...]) + bd2_ref[...], 0.0)
        zm = jnp.clip(jnp.exp(_bmm(e2, wdm_ref[...]) + bdm_ref[...]),
                      1e-5, 1e6)
        out_ref[...] = zm * sf_ref[...]


def kernel(adj, features, size_factors, noise,
           W_gc1, W_gc2, W_mean, W_logstd, W_mul, b_mul,
           W_d1, b_d1, W_d2, b_d2, W_pi, b_pi, W_disp, b_disp, W_dm, b_dm):
    n, f = features.shape
    h1d = W_gc1.shape[1]
    h2d = W_gc2.shape[1]

    bm = 1024
    nrb = n // bm

    g = pl.pallas_call(
        _gcn_kernel,
        grid=(3, nrb),
        in_specs=[
            pl.BlockSpec((bm, n), lambda p, j: (jnp.where(p == 0, j, nrb - 1), 0)),
            pl.BlockSpec((n, f), lambda p, j: (0, 0)),
            pl.BlockSpec((f, h1d), lambda p, j: (0, 0)),
            pl.BlockSpec((h1d, h2d), lambda p, j: (0, 0)),
        ],
        out_specs=pl.BlockSpec((bm, h2d), lambda p, j: (j, 0)),
        out_shape=jax.ShapeDtypeStruct((n, h2d), _F32),
        scratch_shapes=[
            pltpu.VMEM((n, n), _BF),
            pltpu.VMEM((n, h1d), _BF),
            pltpu.VMEM((n, h1d), _BF),
            pltpu.VMEM((n, h2d), _BF),
            pltpu.VMEM((n, h2d), _BF),
        ],
        compiler_params=pltpu.CompilerParams(
            dimension_semantics=("arbitrary", "arbitrary")),
    )(adj, features, W_gc1, W_gc2)

    bn = 512
    nblk = n // bn
    out = pl.pallas_call(
        _mid_kernel,
        grid=(nblk,),
        in_specs=[
            pl.BlockSpec((n, h2d), lambda j: (0, 0)),
            pl.BlockSpec((h2d, bn), lambda j: (0, j)),
            pl.BlockSpec((h2d, bn), lambda j: (0, j)),
            pl.BlockSpec((n, bn), lambda j: (0, j)),
            pl.BlockSpec((n, f), lambda j: (0, 0)),
            pl.BlockSpec((bn, n), lambda j: (j, 0)),
            pl.BlockSpec((1, n), lambda j: (0, 0)),
            pl.BlockSpec((n, h1d), lambda j: (0, 0)),
            pl.BlockSpec((1, h1d), lambda j: (0, 0)),
            pl.BlockSpec((h1d, h2d), lambda j: (0, 0)),
            pl.BlockSpec((1, h2d), lambda j: (0, 0)),
            pl.BlockSpec((h2d, n), lambda j: (0, 0)),
            pl.BlockSpec((1, n), lambda j: (0, 0)),
            pl.BlockSpec((f, 1), lambda j: (0, 0)),
        ],
        out_specs=pl.BlockSpec((f, n), lambda j: (0, 0)),
        out_shape=jax.ShapeDtypeStruct((f, n), _F32),
        scratch_shapes=[
            pltpu.VMEM((f, n), _F32),
            pltpu.VMEM((n, f), _BF),
            pltpu.VMEM((n, h2d), _BF),
            pltpu.VMEM((n, h2d), _BF),
        ],
        compiler_params=pltpu.CompilerParams(
            dimension_semantics=("arbitrary",)),
    )(g, W_logstd, W_mean, noise, features, W_mul,
      b_mul.reshape(1, -1), W_d1, b_d1.reshape(1, -1),
      W_d2, b_d2.reshape(1, -1), W_dm, b_dm.reshape(1, -1),
      size_factors.reshape(-1, 1))
    return out
